# baseline (device time: 110191 ns/iter reference)
import jax
import jax.numpy as jnp
from jax import lax
from jax.experimental import pallas as pl
from jax.experimental.pallas import tpu as pltpu

N_DEV = 4
SQ = 1024
SKV = 1024
H_LOC = 8
DH = 128
D_LOC = H_LOC * DH
D_OUT = 1024
WINDOW = 128
SCALE = 0.08838834764831843


def kernel(x, Wq, K_ext, V_ext, Wo):
    i = lax.axis_index("i")
    x2 = x.reshape(SQ, x.shape[-1]).astype(jnp.bfloat16)
    Wq_i = lax.dynamic_slice_in_dim(Wq, i * D_LOC, D_LOC, axis=1).astype(
        jnp.bfloat16
    )
    Wo_i = lax.dynamic_slice_in_dim(Wo, i * D_LOC, D_LOC, axis=0).astype(
        jnp.bfloat16
    )
    K2 = K_ext.reshape(SKV, D_LOC).astype(jnp.bfloat16)
    V2 = V_ext.reshape(SKV, D_LOC).astype(jnp.bfloat16)

    def body(
        x_ref,
        wq_ref,
        k_ref,
        v_ref,
        wo_ref,
        out_ref,
        ctx_ref,
        comm_ref,
        send_sems,
        recv_sems,
    ):
        my = lax.axis_index("i")
        left = lax.rem(my + (N_DEV - 1), N_DEV)
        right = lax.rem(my + 1, N_DEV)

        q = jnp.dot(
            x_ref[...], wq_ref[...], preferred_element_type=jnp.float32
        )
        qi = lax.broadcasted_iota(jnp.int32, (SQ, SKV), 0)
        ki = lax.broadcasted_iota(jnp.int32, (SQ, SKV), 1)
        mask = jnp.abs(qi - ki) <= WINDOW
        for h in range(H_LOC):
            qh = q[:, h * DH : (h + 1) * DH].astype(jnp.bfloat16)
            kh = k_ref[:, h * DH : (h + 1) * DH]
            vh = v_ref[:, h * DH : (h + 1) * DH]
            s = (
                lax.dot_general(
                    qh,
                    kh,
                    (((1,), (1,)), ((), ())),
                    preferred_element_type=jnp.float32,
                )
                * SCALE
            )
            s = jnp.where(mask, s, -1e9)
            m = jnp.max(s, axis=1, keepdims=True)
            w = jnp.exp(s - m)
            w = w / jnp.sum(w, axis=1, keepdims=True)
            ctx_h = jnp.dot(
                w.astype(jnp.bfloat16), vh, preferred_element_type=jnp.float32
            )
            ctx_ref[:, h * DH : (h + 1) * DH] = ctx_h.astype(jnp.bfloat16)

        partial = jnp.dot(
            ctx_ref[...], wo_ref[...], preferred_element_type=jnp.float32
        )

        barrier_sem = pltpu.get_barrier_semaphore()
        for nbr in [left, right]:
            pl.semaphore_signal(
                barrier_sem,
                inc=1,
                device_id=(nbr,),
                device_id_type=pl.DeviceIdType.MESH,
            )
        pl.semaphore_wait(barrier_sem, 2)

        comm_ref[0] = partial.astype(jnp.bfloat16)
        out_ref[...] = partial
        for hop in range(N_DEV - 1):
            rdma = pltpu.make_async_remote_copy(
                src_ref=comm_ref.at[hop],
                dst_ref=comm_ref.at[hop + 1],
                send_sem=send_sems.at[hop],
                recv_sem=recv_sems.at[hop],
                device_id=(right,),
                device_id_type=pl.DeviceIdType.MESH,
            )
            rdma.start()
            rdma.wait()
            out_ref[...] += comm_ref[hop + 1].astype(jnp.float32)

    out = pl.pallas_call(
        body,
        out_shape=jax.ShapeDtypeStruct((SQ, D_OUT), jnp.float32),
        in_specs=[pl.BlockSpec(memory_space=pltpu.VMEM)] * 5,
        out_specs=pl.BlockSpec(memory_space=pltpu.VMEM),
        scratch_shapes=[
            pltpu.VMEM((SQ, D_LOC), jnp.bfloat16),
            pltpu.VMEM((N_DEV, SQ, D_OUT), jnp.bfloat16),
            pltpu.SemaphoreType.DMA((N_DEV - 1,)),
            pltpu.SemaphoreType.DMA((N_DEV - 1,)),
        ],
        compiler_params=pltpu.CompilerParams(collective_id=0),
    )(x2, Wq_i, K2, V2, Wo_i)
    return out.reshape(1, SQ, D_OUT)


# device time: 63359 ns/iter; 1.7392x vs baseline; 1.7392x over previous
import jax
import jax.numpy as jnp
from jax import lax
from jax.experimental import pallas as pl
from jax.experimental.pallas import tpu as pltpu

N_DEV = 4
SQ = 1024
SKV = 1024
H_LOC = 8
DH = 128
D_LOC = H_LOC * DH
D_OUT = 1024
WINDOW = 128
SCALE = 0.08838834764831843
C = SQ // (2 * N_DEV)
HALF = SQ // 2


def kernel(x, Wq, K_ext, V_ext, Wo):
    i = lax.axis_index("i")
    x2 = x.reshape(SQ, x.shape[-1]).astype(jnp.bfloat16)
    Wq_i = lax.dynamic_slice_in_dim(Wq, i * D_LOC, D_LOC, axis=1).astype(
        jnp.bfloat16
    )
    Wo_i = lax.dynamic_slice_in_dim(Wo, i * D_LOC, D_LOC, axis=0).astype(
        jnp.bfloat16
    )
    K2 = K_ext.reshape(SKV, D_LOC).astype(jnp.bfloat16)
    V2 = V_ext.reshape(SKV, D_LOC).astype(jnp.bfloat16)

    def body(
        x_ref,
        wq_ref,
        k_ref,
        v_ref,
        wo_ref,
        out_ref,
        ctx_ref,
        acc_ref,
        send_cw,
        recv_cw,
        send_ccw,
        recv_ccw,
        cw_send_sems,
        cw_recv_sems,
        ccw_send_sems,
        ccw_recv_sems,
    ):
        my = lax.axis_index("i")
        left = lax.rem(my + (N_DEV - 1), N_DEV)
        right = lax.rem(my + 1, N_DEV)
        f32 = jnp.float32
        bf16 = jnp.bfloat16

        q = jnp.dot(
            x_ref[...], wq_ref[...], preferred_element_type=f32
        )
        qi = lax.broadcasted_iota(jnp.int32, (SQ, SKV), 0)
        ki = lax.broadcasted_iota(jnp.int32, (SQ, SKV), 1)
        mask = jnp.abs(qi - ki) <= WINDOW
        for h in range(H_LOC):
            qh = q[:, h * DH : (h + 1) * DH].astype(bf16)
            kh = k_ref[:, h * DH : (h + 1) * DH]
            vh = v_ref[:, h * DH : (h + 1) * DH]
            s = (
                lax.dot_general(
                    qh,
                    kh,
                    (((1,), (1,)), ((), ())),
                    preferred_element_type=f32,
                )
                * SCALE
            )
            s = jnp.where(mask, s, -1e9)
            m = jnp.max(s, axis=1, keepdims=True)
            w = jnp.exp(s - m)
            w = w / jnp.sum(w, axis=1, keepdims=True)
            ctx_h = jnp.dot(
                w.astype(bf16), vh, preferred_element_type=f32
            )
            ctx_ref[:, h * DH : (h + 1) * DH] = ctx_h.astype(bf16)

        acc_ref[...] = jnp.dot(
            ctx_ref[...], wo_ref[...], preferred_element_type=f32
        )

        barrier_sem = pltpu.get_barrier_semaphore()
        for nbr in [left, right]:
            pl.semaphore_signal(
                barrier_sem,
                inc=1,
                device_id=(nbr,),
                device_id_type=pl.DeviceIdType.MESH,
            )
        pl.semaphore_wait(barrier_sem, 2)

        def cw_rows(c):
            return pl.ds(c * C, C)

        def ccw_rows(c):
            return pl.ds(HALF + c * C, C)

        def start_rdma(src, dst, ssem, rsem, dev):
            r = pltpu.make_async_remote_copy(
                src_ref=src,
                dst_ref=dst,
                send_sem=ssem,
                recv_sem=rsem,
                device_id=(dev,),
                device_id_type=pl.DeviceIdType.MESH,
            )
            r.start()
            return r

        for h in range(N_DEV - 1):
            c_cw = lax.rem(my - h + 2 * N_DEV, N_DEV)
            val_cw = acc_ref[cw_rows(c_cw), :]
            if h > 0:
                val_cw = val_cw + recv_cw[h - 1].astype(f32)
            send_cw[h] = val_cw.astype(bf16)
            r_cw = start_rdma(
                send_cw.at[h], recv_cw.at[h],
                cw_send_sems.at[h], cw_recv_sems.at[h], right,
            )
            c_ccw = lax.rem(my + h, N_DEV)
            val_ccw = acc_ref[ccw_rows(c_ccw), :]
            if h > 0:
                val_ccw = val_ccw + recv_ccw[h - 1].astype(f32)
            send_ccw[h] = val_ccw.astype(bf16)
            r_ccw = start_rdma(
                send_ccw.at[h], recv_ccw.at[h],
                ccw_send_sems.at[h], ccw_recv_sems.at[h], left,
            )
            r_cw.wait()
            r_ccw.wait()

        c_own_cw = lax.rem(my + 1, N_DEV)
        red_cw = (
            acc_ref[cw_rows(c_own_cw), :]
            + recv_cw[N_DEV - 2].astype(f32)
        )
        out_ref[cw_rows(c_own_cw), :] = red_cw
        c_own_ccw = lax.rem(my + N_DEV - 1, N_DEV)
        red_ccw = (
            acc_ref[ccw_rows(c_own_ccw), :]
            + recv_ccw[N_DEV - 2].astype(f32)
        )
        out_ref[ccw_rows(c_own_ccw), :] = red_ccw

        send_cw[N_DEV - 1] = red_cw.astype(bf16)
        send_ccw[N_DEV - 1] = red_ccw.astype(bf16)
        for h in range(N_DEV - 1):
            s = (N_DEV - 1) + h
            src_cw = send_cw.at[N_DEV - 1] if h == 0 else recv_cw.at[s - 1]
            r_cw = start_rdma(
                src_cw, recv_cw.at[s],
                cw_send_sems.at[s], cw_recv_sems.at[s], right,
            )
            src_ccw = send_ccw.at[N_DEV - 1] if h == 0 else recv_ccw.at[s - 1]
            r_ccw = start_rdma(
                src_ccw, recv_ccw.at[s],
                ccw_send_sems.at[s], ccw_recv_sems.at[s], left,
            )
            r_cw.wait()
            r_ccw.wait()
            c_r_cw = lax.rem(my - h + 2 * N_DEV, N_DEV)
            out_ref[cw_rows(c_r_cw), :] = recv_cw[s].astype(f32)
            c_r_ccw = lax.rem(my + h, N_DEV)
            out_ref[ccw_rows(c_r_ccw), :] = recv_ccw[s].astype(f32)

    n_stage = 2 * (N_DEV - 1)
    out = pl.pallas_call(
        body,
        out_shape=jax.ShapeDtypeStruct((SQ, D_OUT), jnp.float32),
        in_specs=[pl.BlockSpec(memory_space=pltpu.VMEM)] * 5,
        out_specs=pl.BlockSpec(memory_space=pltpu.VMEM),
        scratch_shapes=[
            pltpu.VMEM((SQ, D_LOC), jnp.bfloat16),
            pltpu.VMEM((SQ, D_OUT), jnp.float32),
            pltpu.VMEM((N_DEV, C, D_OUT), jnp.bfloat16),
            pltpu.VMEM((n_stage, C, D_OUT), jnp.bfloat16),
            pltpu.VMEM((N_DEV, C, D_OUT), jnp.bfloat16),
            pltpu.VMEM((n_stage, C, D_OUT), jnp.bfloat16),
            pltpu.SemaphoreType.DMA((n_stage,)),
            pltpu.SemaphoreType.DMA((n_stage,)),
            pltpu.SemaphoreType.DMA((n_stage,)),
            pltpu.SemaphoreType.DMA((n_stage,)),
        ],
        compiler_params=pltpu.CompilerParams(collective_id=0),
    )(x2, Wq_i, K2, V2, Wo_i)
    return out.reshape(1, SQ, D_OUT)
